# baseline (speedup 1.0000x reference)
import functools

import jax
import jax.numpy as jnp
from jax import lax
from jax.experimental import pallas as pl
from jax.experimental.pallas import tpu as pltpu
from jax.experimental.pallas import tpu_sc as plsc

_LANES = 16


@functools.cache
def _build(B: int, T: int, V: int, D: int):
    info = plsc.get_sparse_core_info()
    nw = info.num_cores * info.num_subcores
    t_per_w = T // nw
    C = t_per_w // 2
    n_chunks = B * (t_per_w // C)
    mesh = plsc.VectorSubcoreMesh(core_axis_name="c", subcore_axis_name="s")

    @functools.partial(
        pl.kernel,
        mesh=mesh,
        out_type=jax.ShapeDtypeStruct((B * T, D), jnp.float32),
        scratch_types=[
            pltpu.VMEM((B, t_per_w), jnp.int32),
            pltpu.VMEM((t_per_w, D), jnp.float32),
            pltpu.VMEM((C, D), jnp.float32),
            pltpu.VMEM((C, D), jnp.float32),
            pltpu.SemaphoreType.DMA,
            pltpu.SemaphoreType.DMA,
            pltpu.SemaphoreType.DMA,
            pltpu.SemaphoreType.DMA,
        ],
    )
    def emb_kernel(idx_hbm, wte_hbm, wpe_hbm, out_hbm,
                   idx_v, wpe_v, rows0, rows1, g0, g1, s0, s1):
        wid = lax.axis_index("s") * info.num_cores + lax.axis_index("c")
        t0 = wid * t_per_w
        rows = (rows0, rows1)
        gsem = (g0, g1)
        ssem = (s0, s1)

        pltpu.sync_copy(wpe_hbm.at[pl.ds(t0, t_per_w)], wpe_v)
        for b in range(B):
            pltpu.sync_copy(idx_hbm.at[pl.ds(b * T + t0, t_per_w)], idx_v.at[b])

        def chunk_off(c):
            b, h = divmod(c, n_chunks // B)
            return b * T + t0 + h * C, b, h * C

        def start_gather(c):
            buf = c % 2
            _, b, hoff = chunk_off(c)
            return pltpu.async_copy(
                wte_hbm.at[idx_v.at[b, pl.ds(hoff, C)]], rows[buf], gsem[buf])

        gathers = [None, None]
        stores = [None, None]
        gathers[0] = start_gather(0)
        for c in range(n_chunks):
            buf = c % 2
            nxt = (c + 1) % 2
            if c + 1 < n_chunks:
                if stores[nxt] is not None:
                    stores[nxt].wait()
                    stores[nxt] = None
                gathers[nxt] = start_gather(c + 1)
            gathers[buf].wait()
            off, _, hoff = chunk_off(c)
            rbuf = rows[buf]

            def add_row(i, carry):
                for j in range(D // _LANES):
                    sl = pl.ds(j * _LANES, _LANES)
                    plsc.addupdate(rbuf.at[i, sl], wpe_v[hoff + i, sl])
                return carry

            lax.fori_loop(0, C, add_row, 0, unroll=2)
            stores[buf] = pltpu.async_copy(rbuf, out_hbm.at[pl.ds(off, C)], ssem[buf])
        for st in stores:
            if st is not None:
                st.wait()

    return emb_kernel


def kernel(idx, wte, wpe):
    b, t = idx.shape
    v, d = wte.shape
    idx_flat = idx.reshape(b * t).astype(jnp.int32)
    out = _build(b, t, v, d)(idx_flat, wte, wpe)
    return out.reshape(b, t, d)
